# Initial kernel scaffold; baseline (speedup 1.0000x reference)
import jax
import jax.numpy as jnp
from jax.experimental import pallas as pl


def kernel(x, edge_index, edge_attr, batch, Wem1, bem1, Wem2, bem2, We1, be1, Wn1, nb1, g1, b1, We2, be2, Wn2, nb2, g2, b2, We3, be3, Wn3, nb3, g3, b3, Wf1, bf1, Wf2, bf2, Wf3, bf3):
    raise NotImplementedError("write your pallas kernel here")


# speedup vs baseline: 1.0334x; 1.0334x over previous
import functools

import jax
import jax.numpy as jnp
from jax import lax
from jax.experimental import pallas as pl
from jax.experimental.pallas import tpu as pltpu
from jax.experimental.pallas import tpu_sc as plsc

_N = 10000
_NP = 10240
_E = 160000
_EP = 163840
_NG = 64
_EB = 2048
_R = 512
_NBLK = _EP // 128
_BPT = _NBLK // 32
_RPT = _NP // 16

_f32 = jnp.float32


def _pre_body(eat, wem1, bem1, wem2, bem2, we1, be1, we2, be2, we3, be3,
              *outs):
    i = pl.program_id(0)
    a = jnp.maximum(jnp.dot(eat[...], wem1[...],
                            preferred_element_type=_f32) + bem1[...], 0.0)
    ea = jnp.dot(a, wem2[...], preferred_element_type=_f32) + bem2[...]
    rows = lax.broadcasted_iota(jnp.int32, (_EB, 1), 0) + i * _EB
    valid = rows < _E
    e1 = jnp.dot(ea, we1[...], preferred_element_type=_f32) + be1[...]
    outs[0][...] = jnp.where(valid, e1, -1e9)
    e2 = jnp.dot(ea, we2[...], preferred_element_type=_f32) + be2[...]
    e2 = jnp.where(valid, e2, -1e9)
    for c in range(4):
        outs[1 + c][...] = e2[:, c * 128:(c + 1) * 128]
    e3 = jnp.dot(ea, we3[...], preferred_element_type=_f32) + be3[...]
    e3 = jnp.where(valid, e3, -1e9)
    for c in range(8):
        outs[5 + c][...] = e3[:, c * 128:(c + 1) * 128]


def _full_spec(arr):
    nd = arr.ndim
    return pl.BlockSpec(arr.shape, lambda i, _nd=nd: (0,) * _nd)


def _pre_kernel(eap, wem1, bem1, wem2, bem2, we1, be1, we2, be2, we3, be3):
    weights = (wem1, bem1, wem2, bem2, we1, be1, we2, be2, we3, be3)
    return pl.pallas_call(
        _pre_body,
        grid=(_EP // _EB,),
        in_specs=[pl.BlockSpec((_EB, 8), lambda i: (i, 0))] +
                 [_full_spec(w) for w in weights],
        out_specs=[pl.BlockSpec((_EB, 128), lambda i: (i, 0))] +
                  [pl.BlockSpec((_EB, 128), lambda i: (i, 0))] * 12,
        out_shape=[jax.ShapeDtypeStruct((_EP, 128), _f32)] +
                  [jax.ShapeDtypeStruct((_EP, 128), _f32)] * 12,
    )(eap, *weights)


def _k1_body(C, refs):
    hs = refs[0:C]
    a0 = refs[C:2 * C]
    a1 = refs[2 * C:3 * C]
    wn, nb, t_ref, s1_ref, s2_ref = refs[3 * C:]
    i = pl.program_id(0)
    acc = None
    for c in range(C):
        m = jnp.dot(hs[c][...] + a0[c][...] + a1[c][...], wn[c],
                    preferred_element_type=_f32)
        acc = m if acc is None else acc + m
    t = jnp.maximum(acc + nb[...], 0.0)
    rows = lax.broadcasted_iota(jnp.int32, (_R, 1), 0) + i * _R
    t = jnp.where(rows < _N, t, 0.0)
    t_ref[...] = t
    ps = jnp.sum(t, axis=0, keepdims=True)
    psq = jnp.sum(t * t, axis=0, keepdims=True)

    @pl.when(i == 0)
    def _():
        s1_ref[...] = ps
        s2_ref[...] = psq

    @pl.when(i > 0)
    def _():
        s1_ref[...] = s1_ref[...] + ps
        s2_ref[...] = s2_ref[...] + psq


def _k1_kernel(hs, a0s, a1s, wn, nb, dout):
    C = len(hs)
    w = hs[0].shape[1]
    body = functools.partial(lambda C_, *r: _k1_body(C_, r), C)
    return pl.pallas_call(
        body,
        grid=(_NP // _R,),
        in_specs=[pl.BlockSpec((_R, w), lambda i: (i, 0))] * (3 * C) +
                 [_full_spec(wn), _full_spec(nb)],
        out_specs=[pl.BlockSpec((_R, dout), lambda i: (i, 0)),
                   pl.BlockSpec((1, dout), lambda i: (0, 0)),
                   pl.BlockSpec((1, dout), lambda i: (0, 0))],
        out_shape=[jax.ShapeDtypeStruct((_NP, dout), _f32),
                   jax.ShapeDtypeStruct((1, dout), _f32),
                   jax.ShapeDtypeStruct((1, dout), _f32)],
    )(*hs, *a0s, *a1s, wn, nb)


def _k2_body(n_chunks, t, s1, s2, g, b, *outs):
    m = s1[...] / float(_N)
    v = s2[...] / float(_N) - m * m
    scale = lax.rsqrt(v + 1e-5) * g[...]
    y = (t[...] - m) * scale + b[...]
    if n_chunks == 0:
        outs[0][...] = y
    else:
        for c in range(n_chunks):
            outs[c][...] = y[:, c * 128:(c + 1) * 128]


def _k2_kernel(t, s1, s2, g, b, n_chunks):
    dout = t.shape[1]
    if n_chunks == 0:
        out_specs = [pl.BlockSpec((_R, dout), lambda i: (i, 0))]
        out_shape = [jax.ShapeDtypeStruct((_NP, dout), _f32)]
    else:
        out_specs = [pl.BlockSpec((_R, 128), lambda i: (i, 0))] * n_chunks
        out_shape = [jax.ShapeDtypeStruct((_NP, 128), _f32)] * n_chunks
    res = pl.pallas_call(
        functools.partial(_k2_body, n_chunks),
        grid=(_NP // _R,),
        in_specs=[pl.BlockSpec((_R, dout), lambda i: (i, 0)),
                  _full_spec(s1), _full_spec(s2),
                  _full_spec(g), _full_spec(b)],
        out_specs=out_specs,
        out_shape=out_shape,
    )(t, s1, s2, g, b)
    return res


def _pool_body(h3, bt, wf1, bf1, wf2, bf2, wf3, bf3, out_ref, psum, pcnt):
    i = pl.program_id(0)

    @pl.when(i == 0)
    def _():
        psum[...] = jnp.zeros_like(psum)
        pcnt[...] = jnp.zeros_like(pcnt)

    gid = lax.broadcasted_iota(jnp.int32, (_NG, _R), 0)
    oh = jnp.where(gid == bt[0], 1.0, 0.0)
    psum[...] = psum[...] + jnp.dot(oh, h3[...], preferred_element_type=_f32)
    cnt = jnp.sum(oh, axis=1, keepdims=True)
    pcnt[...] = pcnt[...] + lax.broadcast_in_dim(cnt, (_NG, 128), (0, 1))

    @pl.when(i == _NP // _R - 1)
    def _():
        den = jnp.maximum(pcnt[:, 0:1], 1.0)
        pooled = psum[...] / den
        r = jnp.maximum(jnp.dot(pooled, wf1[...],
                                preferred_element_type=_f32) + bf1[...], 0.0)
        r = jnp.maximum(jnp.dot(r, wf2[...],
                                preferred_element_type=_f32) + bf2[...], 0.0)
        out_ref[...] = jnp.dot(r, wf3[...],
                               preferred_element_type=_f32) + bf3[...]


def _pool_kernel(h3, batchp, wf1, bf1, wf2, bf2, wf3, bf3):
    weights = (wf1, bf1, wf2, bf2, wf3, bf3)
    return pl.pallas_call(
        _pool_body,
        grid=(_NP // _R,),
        in_specs=[pl.BlockSpec((_R, 2048), lambda i: (i, 0)),
                  pl.BlockSpec((1, 1, _R), lambda i: (i, 0, 0))] +
                 [_full_spec(w) for w in weights],
        out_specs=pl.BlockSpec((_NG, 128), lambda i: (0, 0)),
        out_shape=jax.ShapeDtypeStruct((_NG, 128), _f32),
        scratch_shapes=[pltpu.VMEM((_NG, 2048), _f32),
                        pltpu.VMEM((_NG, 128), _f32)],
    )(h3, batchp, *weights)


def _make_sc(C, W):
    mesh = plsc.VectorSubcoreMesh(core_axis_name="c", subcore_axis_name="s",
                                  num_cores=2, num_subcores=16)
    out_type = [jax.ShapeDtypeStruct((_NP, W), _f32) for _ in range(2 * C)]
    scratch = [pltpu.VMEM((_BPT, 128), jnp.int32),
               pltpu.VMEM((_BPT, 128), jnp.int32),
               pltpu.VMEM((128, W), _f32),
               pltpu.VMEM((128, W), _f32),
               pltpu.VMEM_SHARED((_NP, W), _f32)]

    def body(*refs):
        tables = refs[0:C]
        es = refs[C:2 * C]
        srcb = refs[2 * C]
        dstb = refs[2 * C + 1]
        zz = refs[2 * C + 2]
        outs = refs[2 * C + 3:2 * C + 3 + 2 * C]
        srcv, dstv, gbuf, ebuf, acc = refs[2 * C + 3 + 2 * C:]

        cid = lax.axis_index("c")
        sid = lax.axis_index("s")
        base = cid * (_NBLK // 2) + sid * _BPT
        pltpu.sync_copy(srcb.at[pl.ds(base, _BPT)], srcv)
        pltpu.sync_copy(dstb.at[pl.ds(base, _BPT)], dstv)
        r0 = sid * _RPT

        for c in range(C):
            pltpu.sync_copy(zz.at[pl.ds(r0, _RPT)], acc.at[pl.ds(r0, _RPT)])
            plsc.subcore_barrier()

            def step(j, carry, _c=c):
                pltpu.sync_copy(tables[_c].at[srcv.at[j]], gbuf)
                blk = base + j
                pltpu.sync_copy(es[_c].at[pl.ds(blk * 128, 128)], ebuf)

                def rowf(r, cc):
                    for k in range(W // 16):
                        sl = pl.ds(k * 16, 16)
                        gbuf[r, sl] = jnp.maximum(gbuf[r, sl] + ebuf[r, sl],
                                                  0.0)
                    return cc

                lax.fori_loop(0, 128, rowf, 0)
                pltpu.sync_copy(gbuf, acc.at[dstv.at[j]], add=True)
                return carry

            lax.fori_loop(0, _BPT, step, 0)
            plsc.subcore_barrier()
            for k in range(2):
                @pl.when(cid == k)
                def _(_c=c, _k=k):
                    pltpu.sync_copy(acc.at[pl.ds(r0, _RPT)],
                                    outs[2 * _c + _k].at[pl.ds(r0, _RPT)])
            plsc.subcore_barrier()

    return pl.kernel(body, out_type=out_type, mesh=mesh,
                     scratch_types=scratch)


def kernel(x, edge_index, edge_attr, batch, Wem1, bem1, Wem2, bem2, We1, be1,
           Wn1, nb1, g1, b1, We2, be2, Wn2, nb2, g2, b2, We3, be3, Wn3, nb3,
           g3, b3, Wf1, bf1, Wf2, bf2, Wf3, bf3):
    i32 = jnp.int32
    xp = jnp.zeros((_NP, 128), _f32).at[:_N, :6].set(x)
    eap = jnp.zeros((_EP, 8), _f32).at[:_E, :6].set(edge_attr)
    src = jnp.zeros((_EP,), i32).at[:_E].set(edge_index[0]).reshape(_NBLK, 128)
    dst = jnp.zeros((_EP,), i32).at[:_E].set(edge_index[1]).reshape(_NBLK, 128)
    batchp = jnp.full((_NP,), _NG, i32).at[:_N].set(batch).reshape(
        _NP // _R, 1, _R)
    z128 = jnp.zeros((_NP, 128), _f32)

    wem1p = jnp.zeros((8, 64), _f32).at[:6].set(Wem1)
    we1p = jnp.zeros((64, 128), _f32).at[:, :6].set(We1)
    be1p = jnp.zeros((1, 128), _f32).at[0, :6].set(be1)
    wn1p = jnp.zeros((1, 128, 512), _f32).at[0, :6].set(Wn1)
    wn2r = Wn2.reshape(4, 128, 1024)
    wn3r = Wn3.reshape(8, 128, 2048)
    wf3p = jnp.zeros((512, 128), _f32).at[:, :86].set(Wf3)
    bf3p = jnp.zeros((1, 128), _f32).at[0, :86].set(bf3)
    r1 = lambda a: a.reshape(1, -1)

    pre = _pre_kernel(eap, wem1p, r1(bem1), Wem2, r1(bem2),
                      we1p, be1p, We2, r1(be2), We3, r1(be3))
    e1, e2c, e3c = pre[0], pre[1:5], pre[5:13]

    a1 = _make_sc(1, 128)(xp, e1, src, dst, z128)
    t1, s11, s12 = _k1_kernel([xp], [a1[0]], [a1[1]], wn1p, r1(nb1), 512)
    h1c = _k2_kernel(t1, s11, s12, r1(g1), r1(b1), 4)

    a2 = _make_sc(4, 128)(*h1c, *e2c, src, dst, z128)
    t2, s21, s22 = _k1_kernel(h1c, a2[0::2], a2[1::2], wn2r, r1(nb2), 1024)
    h2c = _k2_kernel(t2, s21, s22, r1(g2), r1(b2), 8)

    a3 = _make_sc(8, 128)(*h2c, *e3c, src, dst, z128)
    t3, s31, s32 = _k1_kernel(h2c, a3[0::2], a3[1::2], wn3r, r1(nb3), 2048)
    h3 = _k2_kernel(t3, s31, s32, r1(g3), r1(b3), 0)[0]

    outp = _pool_kernel(h3, batchp, Wf1, r1(bf1), Wf2, r1(bf2), wf3p, bf3p)
    return outp[:, :86]
